# baseline (speedup 1.0000x reference)
import jax
import jax.numpy as jnp
from jax import lax
from jax.experimental import pallas as pl
from jax.experimental.pallas import tpu as pltpu
from jax.experimental.pallas import tpu_sc as plsc

D_MODEL = 128
SCALE = D_MODEL ** 0.5
NUM_CORES = 2
NUM_SUBCORES = 16
NUM_WORKERS = NUM_CORES * NUM_SUBCORES
BATCH = 4096
SEQ = 50
TOTAL_ROWS = BATCH * SEQ
ROWS_PER_WORKER = TOTAL_ROWS // NUM_WORKERS
CHUNK = 128
NCHUNK = ROWS_PER_WORKER // CHUNK
LANES = 16
NBUF = 3


def _emb_body(x_hbm, lut_hbm, out_hbm, idx_v, in0, in1, in2, o0, o1, o2,
              g0, g1, g2, s0, s1, s2):
    ins = (in0, in1, in2)
    outs = (o0, o1, o2)
    gsems = (g0, g1, g2)
    ssems = (s0, s1, s2)
    wid = lax.axis_index("s") * NUM_CORES + lax.axis_index("c")
    base = wid * ROWS_PER_WORKER

    pltpu.sync_copy(x_hbm.at[:, pl.ds(wid * CHUNK, CHUNK)], idx_v)

    for b in range(NBUF):
        pltpu.make_async_copy(lut_hbm.at[idx_v.at[b]], ins[b], gsems[b]).start()

    @pl.loop(0, NBUF * pl.cdiv(NCHUNK, NBUF), step=NBUF)
    def _chunks(g):
        for b in range(NBUF):
            j = g + b

            @pl.when(j < NCHUNK)
            def _body():
                pltpu.make_async_copy(
                    lut_hbm.at[idx_v.at[b]], ins[b], gsems[b]).wait()

                @pl.when(j >= NBUF)
                def _drain():
                    pltpu.make_async_copy(
                        outs[b], out_hbm.at[pl.ds(base, CHUNK)],
                        ssems[b]).wait()

                @plsc.parallel_loop(0, CHUNK, unroll=2)
                def _scale(r):
                    for c in range(D_MODEL // LANES):
                        sl = pl.ds(c * LANES, LANES)
                        outs[b][r, sl] = ins[b][r, sl] * jnp.float32(SCALE)

                @pl.when(j + NBUF < NCHUNK)
                def _refill():
                    pltpu.make_async_copy(
                        lut_hbm.at[idx_v.at[j + NBUF]], ins[b],
                        gsems[b]).start()

                pltpu.make_async_copy(
                    outs[b], out_hbm.at[pl.ds(base + j * CHUNK, CHUNK)],
                    ssems[b]).start()

    for b in range(NBUF):
        pltpu.make_async_copy(
            outs[b], out_hbm.at[pl.ds(base, CHUNK)], ssems[b]).wait()


_emb = pl.kernel(
    _emb_body,
    out_type=jax.ShapeDtypeStruct((TOTAL_ROWS, D_MODEL), jnp.float32),
    mesh=plsc.VectorSubcoreMesh(core_axis_name="c", subcore_axis_name="s"),
    compiler_params=pltpu.CompilerParams(use_tc_tiling_on_sc=True),
    scratch_types=(
        [pltpu.VMEM((NCHUNK, CHUNK), jnp.int32)]
        + [pltpu.VMEM((CHUNK, D_MODEL), jnp.float32)] * NBUF
        + [pltpu.VMEM((CHUNK, D_MODEL), jnp.float32)] * NBUF
        + [pltpu.SemaphoreType.DMA] * (2 * NBUF)
    ),
)


@jax.jit
def kernel(x, lut):
    xt = jnp.swapaxes(x.astype(jnp.int32), 0, 1)
    out = _emb(xt, lut)
    return jnp.swapaxes(out.reshape(SEQ, BATCH, D_MODEL), 0, 1)


# speedup vs baseline: 1.0021x; 1.0021x over previous
import jax
import jax.numpy as jnp
from jax import lax
from jax.experimental import pallas as pl
from jax.experimental.pallas import tpu as pltpu
from jax.experimental.pallas import tpu_sc as plsc

D_MODEL = 128
SCALE = D_MODEL ** 0.5
NUM_CORES = 2
NUM_SUBCORES = 16
NUM_WORKERS = NUM_CORES * NUM_SUBCORES
BATCH = 4096
SEQ = 50
TOTAL_ROWS = BATCH * SEQ
ROWS_PER_WORKER = TOTAL_ROWS // NUM_WORKERS
CHUNK = 128
NCHUNK = ROWS_PER_WORKER // CHUNK
LANES = 16


NBUF = 3


def _emb_body(x_hbm, lut_hbm, out_hbm, idx_v, in0, in1, in2, o0, o1, o2,
              g0, g1, g2, s0, s1, s2):
    ins = (in0, in1, in2)
    outs = (o0, o1, o2)
    gsems = (g0, g1, g2)
    ssems = (s0, s1, s2)
    wid = lax.axis_index("s") * NUM_CORES + lax.axis_index("c")
    base = wid * ROWS_PER_WORKER

    pltpu.sync_copy(x_hbm.at[wid], idx_v)

    for b in range(NBUF):
        pltpu.make_async_copy(lut_hbm.at[idx_v.at[b]], ins[b], gsems[b]).start()

    @pl.loop(0, NBUF * pl.cdiv(NCHUNK, NBUF), step=NBUF)
    def _chunks(g):
        for b in range(NBUF):
            j = g + b

            @pl.when(j < NCHUNK)
            def _body():
                pltpu.make_async_copy(
                    lut_hbm.at[idx_v.at[b]], ins[b], gsems[b]).wait()

                @pl.when(j >= NBUF)
                def _drain():
                    pltpu.make_async_copy(
                        outs[b], out_hbm.at[pl.ds(base, CHUNK)],
                        ssems[b]).wait()

                @plsc.parallel_loop(0, CHUNK, unroll=2)
                def _scale(r):
                    for c in range(D_MODEL // LANES):
                        sl = pl.ds(c * LANES, LANES)
                        outs[b][r, sl] = ins[b][r, sl] * jnp.float32(SCALE)

                @pl.when(j + NBUF < NCHUNK)
                def _refill():
                    pltpu.make_async_copy(
                        lut_hbm.at[idx_v.at[j + NBUF]], ins[b],
                        gsems[b]).start()

                pltpu.make_async_copy(
                    outs[b], out_hbm.at[pl.ds(base + j * CHUNK, CHUNK)],
                    ssems[b]).start()

    for b in range(NBUF):
        pltpu.make_async_copy(
            outs[b], out_hbm.at[pl.ds(base, CHUNK)], ssems[b]).wait()


_emb = pl.kernel(
    _emb_body,
    out_type=jax.ShapeDtypeStruct((TOTAL_ROWS, D_MODEL), jnp.float32),
    mesh=plsc.VectorSubcoreMesh(core_axis_name="c", subcore_axis_name="s"),
    scratch_types=(
        [pltpu.VMEM((NCHUNK, CHUNK), jnp.int32)]
        + [pltpu.VMEM((CHUNK, D_MODEL), jnp.float32)] * NBUF
        + [pltpu.VMEM((CHUNK, D_MODEL), jnp.float32)] * NBUF
        + [pltpu.SemaphoreType.DMA] * (2 * NBUF)
    ),
)


@jax.jit
def kernel(x, lut):
    xt = jnp.swapaxes(x.astype(jnp.int32), 0, 1)
    xr = xt.reshape(NUM_WORKERS, NCHUNK, CHUNK)
    out = _emb(xr, lut)
    return jnp.swapaxes(out.reshape(SEQ, BATCH, D_MODEL), 0, 1)
